# baseline (speedup 1.0000x reference)
import functools

import jax
import jax.numpy as jnp
from jax import lax
from jax.experimental import pallas as pl
from jax.experimental.pallas import tpu as pltpu
from jax.experimental.pallas import tpu_sc as plsc

B, F, V, D = 4, 100000, 50000, 128
TOTAL = B * F
NC, NS = 2, 16
NW = NC * NS
WPB = NW // B
PW = F // WPB
PWE = PW + 4
PWO = PW - 4
C = 128
NCH = PWO // C
TAILE = PWE - NCH * C
TAILO = PWO - NCH * C


def _sc_gather(fs_flat, x):
    mesh = plsc.VectorSubcoreMesh(core_axis_name="c", subcore_axis_name="s")

    @functools.partial(
        pl.kernel,
        mesh=mesh,
        out_type=jax.ShapeDtypeStruct((TOTAL, D), jnp.float32),
        scratch_types=[
            pltpu.VMEM((PWE,), jnp.int32),
            [pltpu.VMEM((C, D), jnp.float32) for _ in range(4)],
            [pltpu.SemaphoreType.DMA for _ in range(4)],
            [pltpu.SemaphoreType.DMA for _ in range(4)],
        ],
    )
    def k(fs_hbm, x_hbm, out_hbm, idx_v, rows, gsem, osem):
        wid = lax.axis_index("s") * NC + lax.axis_index("c")
        bid = wid // WPB
        j = wid % WPB
        par = j % 2
        start = pl.multiple_of(bid * F + j * PW + 4 * par, 8)
        even = par == 0

        pltpu.sync_copy(fs_hbm.at[pl.ds(start, PWO)], idx_v.at[pl.ds(0, PWO)])

        @pl.when(even)
        def _():
            pltpu.sync_copy(
                fs_hbm.at[pl.ds(start + PWO, 8)], idx_v.at[pl.ds(PWO, 8)]
            )

        def fire(c, s, n):
            pltpu.async_copy(
                x_hbm.at[bid].at[idx_v.at[pl.ds(c * C, n)]],
                rows[s].at[pl.ds(0, n)],
                gsem[s],
            )

        def drain_gather(s, n):
            pltpu.make_async_copy(
                x_hbm.at[0].at[pl.ds(0, n)],
                rows[s].at[pl.ds(0, n)],
                gsem[s],
            ).wait()

        def out_slice(c, n):
            return out_hbm.at[pl.ds(start + c * C, n)]

        def fire_out(c, s, n):
            pltpu.async_copy(rows[s].at[pl.ds(0, n)], out_slice(c, n), osem[s])

        def drain_out(c, s, n):
            pltpu.make_async_copy(
                rows[s].at[pl.ds(0, n)], out_slice(c, n), osem[s]
            ).wait()

        def step(c, s, s_prev, first, wait_slot):
            if wait_slot:
                drain_out(c - 4, s, C)
            fire(c, s, C)
            if not first:
                drain_gather(s_prev, C)
                fire_out(c - 1, s_prev, C)

        def body(t, carry):
            c0 = 4 * t

            @pl.when(t == 0)
            def _():
                step(0, 0, 3, True, False)
                step(1, 1, 0, False, False)
                step(2, 2, 1, False, False)
                step(3, 3, 2, False, False)

            @pl.when(t > 0)
            def _():
                step(c0, 0, 3, False, True)
                step(c0 + 1, 1, 0, False, True)
                step(c0 + 2, 2, 1, False, True)
                step(c0 + 3, 3, 2, False, True)

            return carry

        lax.fori_loop(0, 24, body, 0)

        drain_out(92, 0, C)
        fire(96, 0, C)
        drain_gather(3, C)
        fire_out(95, 3, C)

        drain_out(93, 1, C)

        def tail(n):
            fire(NCH, 1, n)
            drain_gather(0, C)
            fire_out(96, 0, C)
            drain_gather(1, n)
            fire_out(NCH, 1, n)
            drain_out(94, 2, C)
            drain_out(95, 3, C)
            drain_out(96, 0, C)
            drain_out(NCH, 1, n)

        @pl.when(even)
        def _():
            tail(TAILE)

        @pl.when(jnp.logical_not(even))
        def _():
            tail(TAILO)

    return k(fs_flat, x)


def kernel(fs, x):
    fs_flat = fs.reshape(TOTAL).astype(jnp.int32)
    out = _sc_gather(fs_flat, x)
    return out.reshape(B, F, D)


# speedup vs baseline: 1.0034x; 1.0034x over previous
import functools

import jax
import jax.numpy as jnp
from jax import lax
from jax.experimental import pallas as pl
from jax.experimental.pallas import tpu as pltpu
from jax.experimental.pallas import tpu_sc as plsc

B, F, V, D = 4, 100000, 50000, 128
TOTAL = B * F
NC, NS = 2, 16
NW = NC * NS
WPB = NW // B
PW = F // WPB
PWE = PW + 4
PWO = PW - 4
C = 128
NCH = PWO // C
TAILE = PWE - NCH * C
TAILO = PWO - NCH * C


def _sc_gather(fs_flat, x):
    mesh = plsc.VectorSubcoreMesh(core_axis_name="c", subcore_axis_name="s")

    @functools.partial(
        pl.kernel,
        mesh=mesh,
        out_type=jax.ShapeDtypeStruct((TOTAL, D), jnp.float32),
        scratch_types=[
            pltpu.VMEM((PWE,), jnp.int32),
            [pltpu.VMEM((C, D), jnp.float32) for _ in range(3)],
            [pltpu.SemaphoreType.DMA for _ in range(3)],
            [pltpu.SemaphoreType.DMA for _ in range(3)],
        ],
    )
    def k(fs_hbm, x_hbm, out_hbm, idx_v, rows, gsem, osem):
        wid = lax.axis_index("s") * NC + lax.axis_index("c")
        bid = wid // WPB
        j = wid % WPB
        par = j % 2
        start = pl.multiple_of(bid * F + j * PW + 4 * par, 8)
        even = par == 0

        pltpu.sync_copy(fs_hbm.at[pl.ds(start, PWO)], idx_v.at[pl.ds(0, PWO)])

        @pl.when(even)
        def _():
            pltpu.sync_copy(
                fs_hbm.at[pl.ds(start + PWO, 8)], idx_v.at[pl.ds(PWO, 8)]
            )

        def fire(c, s, n):
            pltpu.async_copy(
                x_hbm.at[bid].at[idx_v.at[pl.ds(c * C, n)]],
                rows[s].at[pl.ds(0, n)],
                gsem[s],
            )

        def drain_gather(s, n):
            pltpu.make_async_copy(
                x_hbm.at[0].at[pl.ds(0, n)],
                rows[s].at[pl.ds(0, n)],
                gsem[s],
            ).wait()

        def out_slice(c, n):
            return out_hbm.at[pl.ds(start + c * C, n)]

        def fire_out(c, s, n):
            pltpu.async_copy(rows[s].at[pl.ds(0, n)], out_slice(c, n), osem[s])

        def drain_out(c, s, n):
            pltpu.make_async_copy(
                rows[s].at[pl.ds(0, n)], out_slice(c, n), osem[s]
            ).wait()

        def step(c, s, s_prev, first, wait_slot):
            if wait_slot:
                drain_out(c - 3, s, C)
            fire(c, s, C)
            if not first:
                drain_gather(s_prev, C)
                fire_out(c - 1, s_prev, C)

        def body(t, carry):
            c0 = 3 * t

            @pl.when(t == 0)
            def _():
                step(0, 0, 2, True, False)
                step(1, 1, 0, False, False)
                step(2, 2, 1, False, False)

            @pl.when(t > 0)
            def _():
                step(c0, 0, 2, False, True)
                step(c0 + 1, 1, 0, False, True)
                step(c0 + 2, 2, 1, False, True)

            return carry

        lax.fori_loop(0, 32, body, 0)

        drain_out(93, 0, C)
        fire(96, 0, C)
        drain_gather(2, C)
        fire_out(95, 2, C)

        drain_out(94, 1, C)

        def tail(n):
            fire(NCH, 1, n)
            drain_gather(0, C)
            fire_out(96, 0, C)
            drain_gather(1, n)
            fire_out(NCH, 1, n)
            drain_out(95, 2, C)
            drain_out(96, 0, C)
            drain_out(NCH, 1, n)

        @pl.when(even)
        def _():
            tail(TAILE)

        @pl.when(jnp.logical_not(even))
        def _():
            tail(TAILO)

    return k(fs_flat, x)


def kernel(fs, x):
    fs_flat = fs.reshape(TOTAL).astype(jnp.int32)
    out = _sc_gather(fs_flat, x)
    return out.reshape(B, F, D)
